# baseline (speedup 1.0000x reference)
import functools

import jax
import jax.numpy as jnp
from jax import lax
from jax.experimental import pallas as pl
from jax.experimental.pallas import tpu as pltpu
from jax.experimental.pallas import tpu_sc as plsc

N = 10000
E = 320000
F_IN = 128
H = 16
C = 40

NTILES = 32
CH = 2000
GRP = CH // 16

_SC_PARAMS = pltpu.CompilerParams(needs_layout_passes=False)
_MESH = plsc.VectorSubcoreMesh(core_axis_name="c", subcore_axis_name="s")


def _wid():
    return lax.axis_index("s") * 2 + lax.axis_index("c")


def _zero(ref):
    @plsc.parallel_loop(0, N // 16, unroll=8)
    def _(i):
        ref[pl.ds(i * 16, 16)] = jnp.zeros((16,), jnp.float32)


def _edge_bufs(with_row=True):
    per_par = ([pltpu.VMEM((CH,), jnp.int32)] if with_row else []) + [
        pltpu.VMEM((CH,), jnp.int32),
        pltpu.VMEM((CH,), jnp.float32),
        pltpu.SemaphoreType.DMA,
    ]
    return per_par + per_par


def _pipelined_chunks(nch, start_fn, wait_fn, proc_fn):
    start_fn(0, 0)

    def pair(p, _):
        e = 2 * p
        start_fn(e + 1, 1)
        wait_fn(0)
        proc_fn(e, 0)

        @pl.when(e + 2 < nch)
        def _b():
            start_fn(e + 2, 0)

        wait_fn(1)
        proc_fn(e + 1, 1)
        return _

    lax.fori_loop(0, nch // 2, pair, 0)
    if nch % 2:
        wait_fn(0)
        proc_fn(nch - 1, 0)


@functools.partial(
    pl.kernel, mesh=_MESH,
    out_type=jax.ShapeDtypeStruct((NTILES, N), jnp.float32),
    scratch_types=[pltpu.VMEM((N,), jnp.float32)] + _edge_bufs(with_row=False),
    compiler_params=_SC_PARAMS,
)
def _deg_kernel(eif_hbm, ew_hbm, degp_hbm, acc, *bufs):
    colv = (bufs[0], bufs[3])
    eww = (bufs[1], bufs[4])
    sem = (bufs[2], bufs[5])
    wid = _wid()
    _zero(acc)
    epw = E // NTILES
    nch = epw // CH

    def start(ch, b):
        base = wid * epw + ch * CH
        pltpu.async_copy(eif_hbm.at[pl.ds(E + base, CH)], colv[b], sem[b])
        pltpu.async_copy(ew_hbm.at[pl.ds(base, CH)], eww[b], sem[b])

    def wait(b):
        pltpu.make_async_copy(eif_hbm.at[pl.ds(0, CH)], colv[b], sem[b]).wait()
        pltpu.make_async_copy(ew_hbm.at[pl.ds(0, CH)], eww[b], sem[b]).wait()

    def proc(ch, b):
        @plsc.parallel_loop(0, GRP, unroll=8)
        def _g(g):
            cvec = colv[b][pl.ds(g * 16, 16)]
            wvec = eww[b][pl.ds(g * 16, 16)]
            plsc.addupdate_scatter(acc, [cvec], wvec)

    _pipelined_chunks(nch, start, wait, proc)
    pltpu.sync_copy(acc, degp_hbm.at[wid])


@functools.partial(
    pl.kernel, mesh=_MESH,
    out_type=jax.ShapeDtypeStruct((4, H, N), jnp.float32),
    scratch_types=[
        pltpu.VMEM((N,), jnp.int32),
        pltpu.VMEM((N,), jnp.float32),
        pltpu.VMEM((N,), jnp.float32),
    ] + _edge_bufs(),
    compiler_params=_SC_PARAMS,
)
def _l1_kernel(eif_hbm, ew_hbm, pk1t_hbm, s1p_hbm,
               pkcol, acc0, acc1, *bufs):
    rowv = (bufs[0], bufs[4])
    colv = (bufs[1], bufs[5])
    eww = (bufs[2], bufs[6])
    sem = (bufs[3], bufs[7])
    wid = _wid()
    pair = wid // 4
    q = wid % 4
    epq = E // 4
    nch = epq // CH
    _zero(acc0)
    _zero(acc1)
    pltpu.sync_copy(pk1t_hbm.at[pair], pkcol)

    def start(ch, b):
        base = q * epq + ch * CH
        pltpu.async_copy(eif_hbm.at[pl.ds(base, CH)], rowv[b], sem[b])
        pltpu.async_copy(eif_hbm.at[pl.ds(E + base, CH)], colv[b], sem[b])
        pltpu.async_copy(ew_hbm.at[pl.ds(base, CH)], eww[b], sem[b])

    def wait(b):
        pltpu.make_async_copy(eif_hbm.at[pl.ds(0, CH)], rowv[b], sem[b]).wait()
        pltpu.make_async_copy(eif_hbm.at[pl.ds(0, CH)], colv[b], sem[b]).wait()
        pltpu.make_async_copy(ew_hbm.at[pl.ds(0, CH)], eww[b], sem[b]).wait()

    def proc(ch, b):
        @plsc.parallel_loop(0, GRP, unroll=8)
        def _g(g):
            rvec = rowv[b][pl.ds(g * 16, 16)]
            cvec = colv[b][pl.ds(g * 16, 16)]
            wvec = eww[b][pl.ds(g * 16, 16)]
            word = plsc.load_gather(pkcol, [rvec])
            v0 = plsc.bitcast(lax.shift_left(word, 16), jnp.float32)
            v1 = plsc.bitcast(word & jnp.int32(-65536), jnp.float32)
            plsc.addupdate_scatter(acc0, [cvec], v0 * wvec)
            plsc.addupdate_scatter(acc1, [cvec], v1 * wvec)

    _pipelined_chunks(nch, start, wait, proc)
    pltpu.sync_copy(acc0, s1p_hbm.at[q, pair])
    pltpu.sync_copy(acc1, s1p_hbm.at[q, pair + H // 2])


NF2 = 5
NQ2 = 4

@functools.partial(
    pl.kernel, mesh=_MESH,
    out_type=jax.ShapeDtypeStruct((NQ2, C, N), jnp.float32),
    scratch_types=(
        [pltpu.VMEM((N,), jnp.float32) for _ in range(2 * NF2)]
        + _edge_bufs()
    ),
    compiler_params=_SC_PARAMS,
)
def _l2_kernel(eif_hbm, ew_hbm, g2t_hbm, s2p_hbm, *scratch):
    hcols = scratch[:NF2]
    accs = scratch[NF2:2 * NF2]
    bufs = scratch[2 * NF2:]
    rowv = (bufs[0], bufs[4])
    colv = (bufs[1], bufs[5])
    eww = (bufs[2], bufs[6])
    sem = (bufs[3], bufs[7])
    wid = _wid()
    grp_id = wid // NQ2
    q = wid % NQ2
    epq = E // NQ2
    nch = epq // CH
    for k in range(NF2):
        _zero(accs[k])
        pltpu.sync_copy(g2t_hbm.at[grp_id * NF2 + k], hcols[k])

    def start(ch, b):
        base = q * epq + ch * CH
        pltpu.async_copy(eif_hbm.at[pl.ds(base, CH)], rowv[b], sem[b])
        pltpu.async_copy(eif_hbm.at[pl.ds(E + base, CH)], colv[b], sem[b])
        pltpu.async_copy(ew_hbm.at[pl.ds(base, CH)], eww[b], sem[b])

    def wait(b):
        pltpu.make_async_copy(eif_hbm.at[pl.ds(0, CH)], rowv[b], sem[b]).wait()
        pltpu.make_async_copy(eif_hbm.at[pl.ds(0, CH)], colv[b], sem[b]).wait()
        pltpu.make_async_copy(ew_hbm.at[pl.ds(0, CH)], eww[b], sem[b]).wait()

    def proc(ch, b):
        @plsc.parallel_loop(0, GRP, unroll=8)
        def _g(g):
            rvec = rowv[b][pl.ds(g * 16, 16)]
            cvec = colv[b][pl.ds(g * 16, 16)]
            wvec = eww[b][pl.ds(g * 16, 16)]
            for k in range(NF2):
                vals = plsc.load_gather(hcols[k], [rvec]) * wvec
                plsc.addupdate_scatter(accs[k], [cvec], vals)

    _pipelined_chunks(nch, start, wait, proc)
    for k in range(NF2):
        pltpu.sync_copy(accs[k], s2p_hbm.at[q, grp_id * NF2 + k])


def _tc1_body(x_ref, w1_ref, degp_ref, pk1t_ref, z1t_ref, dis_ref, inv_ref):
    deg = 1.0 + jnp.sum(degp_ref[...], axis=0, keepdims=True)
    dis = jnp.where(deg > 0.0, lax.rsqrt(deg), 0.0)
    inv = jnp.where(deg > 0.0, 1.0 / deg, 0.0)
    h1t = lax.dot_general(w1_ref[...], x_ref[...],
                          (((0,), (1,)), ((), ())),
                          preferred_element_type=jnp.float32)
    g1t = h1t * dis
    lo = lax.bitcast_convert_type(g1t[:H // 2].astype(jnp.bfloat16),
                                  jnp.uint16).astype(jnp.uint32)
    hi = lax.bitcast_convert_type(g1t[H // 2:].astype(jnp.bfloat16),
                                  jnp.uint16).astype(jnp.uint32)
    pk1t_ref[...] = lax.bitcast_convert_type(lo | (hi << 16), jnp.int32)
    z1t_ref[...] = h1t * inv
    dis_ref[...] = dis
    inv_ref[...] = inv


def _tc2_body(s1p_ref, z1t_ref, dis_ref, inv_ref, w2_ref, b1_ref,
              x1_ref, g2t_ref, z2t_ref):
    dis = dis_ref[...]
    x1t = jnp.maximum(dis * (s1p_ref[0] + s1p_ref[1] + s1p_ref[2]
                             + s1p_ref[3]) + z1t_ref[...]
                      + b1_ref[...], 0.0)
    h2t = lax.dot_general(w2_ref[...], x1t,
                          (((0,), (0,)), ((), ())),
                          preferred_element_type=jnp.float32)
    x1_ref[...] = x1t.T
    g2t_ref[...] = h2t * dis
    z2t_ref[...] = h2t * inv_ref[...]


def _tc3_body(s2p_ref, z2t_ref, dis_ref, b2_ref, out_ref):
    s = s2p_ref[0] + s2p_ref[1] + s2p_ref[2] + s2p_ref[3]
    out_ref[...] = (dis_ref[...] * s + z2t_ref[...] + b2_ref[...]).T


_tc1 = pl.pallas_call(
    _tc1_body,
    out_shape=[
        jax.ShapeDtypeStruct((H // 2, N), jnp.int32),
        jax.ShapeDtypeStruct((H, N), jnp.float32),
        jax.ShapeDtypeStruct((1, N), jnp.float32),
        jax.ShapeDtypeStruct((1, N), jnp.float32),
    ],
)

_tc2 = pl.pallas_call(
    _tc2_body,
    out_shape=[
        jax.ShapeDtypeStruct((N, H), jnp.float32),
        jax.ShapeDtypeStruct((C, N), jnp.float32),
        jax.ShapeDtypeStruct((C, N), jnp.float32),
    ],
)

_tc3 = pl.pallas_call(
    _tc3_body,
    out_shape=jax.ShapeDtypeStruct((N, C), jnp.float32),
)


@jax.jit
def kernel(x, edge_index, edge_weight, W1, b1, W2, b2):
    eif = edge_index.reshape(-1)
    degp = _deg_kernel(eif, edge_weight)
    pk1t, z1t, dis, inv = _tc1(x, W1, degp)
    s1p = _l1_kernel(eif, edge_weight, pk1t)
    x1, g2t, z2t = _tc2(s1p, z1t, dis, inv, W2, b1[:, None])
    s2p = _l2_kernel(eif, edge_weight, g2t)
    out = _tc3(s2p, z2t, dis, b2[:, None])
    return (out, x1)


# speedup vs baseline: 1.0293x; 1.0293x over previous
import functools

import jax
import jax.numpy as jnp
from jax import lax
from jax.experimental import pallas as pl
from jax.experimental.pallas import tpu as pltpu
from jax.experimental.pallas import tpu_sc as plsc

N = 10000
E = 320000
F_IN = 128
H = 16
C = 40

NTILES = 32
CH = 2000
GRP = CH // 16

_SC_PARAMS = pltpu.CompilerParams(needs_layout_passes=False)
_MESH = plsc.VectorSubcoreMesh(core_axis_name="c", subcore_axis_name="s")


def _wid():
    return lax.axis_index("s") * 2 + lax.axis_index("c")


def _zero(ref):
    @plsc.parallel_loop(0, N // 16, unroll=8)
    def _(i):
        ref[pl.ds(i * 16, 16)] = jnp.zeros((16,), jnp.float32)


def _edge_bufs(with_row=True):
    per_par = ([pltpu.VMEM((CH,), jnp.int32)] if with_row else []) + [
        pltpu.VMEM((CH,), jnp.int32),
        pltpu.VMEM((CH,), jnp.float32),
        pltpu.SemaphoreType.DMA,
    ]
    return per_par + per_par


def _pipelined_chunks(nch, start_fn, wait_fn, proc_fn):
    start_fn(0, 0)

    def pair(p, _):
        e = 2 * p
        start_fn(e + 1, 1)
        wait_fn(0)
        proc_fn(e, 0)

        @pl.when(e + 2 < nch)
        def _b():
            start_fn(e + 2, 0)

        wait_fn(1)
        proc_fn(e + 1, 1)
        return _

    lax.fori_loop(0, nch // 2, pair, 0)
    if nch % 2:
        wait_fn(0)
        proc_fn(nch - 1, 0)


@functools.partial(
    pl.kernel, mesh=_MESH,
    out_type=jax.ShapeDtypeStruct((NTILES, N), jnp.float32),
    scratch_types=[pltpu.VMEM((N,), jnp.float32)] + _edge_bufs(with_row=False),
    compiler_params=_SC_PARAMS,
)
def _deg_kernel(eif_hbm, ew_hbm, degp_hbm, acc, *bufs):
    colv = (bufs[0], bufs[3])
    eww = (bufs[1], bufs[4])
    sem = (bufs[2], bufs[5])
    wid = _wid()
    _zero(acc)
    epw = E // NTILES
    nch = epw // CH

    def start(ch, b):
        base = wid * epw + ch * CH
        pltpu.async_copy(eif_hbm.at[pl.ds(E + base, CH)], colv[b], sem[b])
        pltpu.async_copy(ew_hbm.at[pl.ds(base, CH)], eww[b], sem[b])

    def wait(b):
        pltpu.make_async_copy(eif_hbm.at[pl.ds(0, CH)], colv[b], sem[b]).wait()
        pltpu.make_async_copy(ew_hbm.at[pl.ds(0, CH)], eww[b], sem[b]).wait()

    def proc(ch, b):
        @plsc.parallel_loop(0, GRP, unroll=8)
        def _g(g):
            cvec = colv[b][pl.ds(g * 16, 16)]
            wvec = eww[b][pl.ds(g * 16, 16)]
            plsc.addupdate_scatter(acc, [cvec], wvec)

    _pipelined_chunks(nch, start, wait, proc)
    pltpu.sync_copy(acc, degp_hbm.at[wid])


@functools.partial(
    pl.kernel, mesh=_MESH,
    out_type=jax.ShapeDtypeStruct((4, H, N), jnp.float32),
    scratch_types=[
        pltpu.VMEM((N,), jnp.int32),
        pltpu.VMEM((N,), jnp.float32),
        pltpu.VMEM((N,), jnp.float32),
    ] + _edge_bufs(),
    compiler_params=_SC_PARAMS,
)
def _l1_kernel(eif_hbm, ew_hbm, pk1t_hbm, s1p_hbm,
               pkcol, acc0, acc1, *bufs):
    rowv = (bufs[0], bufs[4])
    colv = (bufs[1], bufs[5])
    eww = (bufs[2], bufs[6])
    sem = (bufs[3], bufs[7])
    wid = _wid()
    pair = wid // 4
    q = wid % 4
    epq = E // 4
    nch = epq // CH
    _zero(acc0)
    _zero(acc1)
    pltpu.sync_copy(pk1t_hbm.at[pair], pkcol)

    def start(ch, b):
        base = q * epq + ch * CH
        pltpu.async_copy(eif_hbm.at[pl.ds(base, CH)], rowv[b], sem[b])
        pltpu.async_copy(eif_hbm.at[pl.ds(E + base, CH)], colv[b], sem[b])
        pltpu.async_copy(ew_hbm.at[pl.ds(base, CH)], eww[b], sem[b])

    def wait(b):
        pltpu.make_async_copy(eif_hbm.at[pl.ds(0, CH)], rowv[b], sem[b]).wait()
        pltpu.make_async_copy(eif_hbm.at[pl.ds(0, CH)], colv[b], sem[b]).wait()
        pltpu.make_async_copy(ew_hbm.at[pl.ds(0, CH)], eww[b], sem[b]).wait()

    def proc(ch, b):
        @plsc.parallel_loop(0, GRP, unroll=8)
        def _g(g):
            rvec = rowv[b][pl.ds(g * 16, 16)]
            cvec = colv[b][pl.ds(g * 16, 16)]
            wvec = eww[b][pl.ds(g * 16, 16)]
            word = plsc.load_gather(pkcol, [rvec])
            v0 = plsc.bitcast(lax.shift_left(word, 16), jnp.float32)
            v1 = plsc.bitcast(word & jnp.int32(-65536), jnp.float32)
            plsc.addupdate_scatter(acc0, [cvec], v0 * wvec)
            plsc.addupdate_scatter(acc1, [cvec], v1 * wvec)

    _pipelined_chunks(nch, start, wait, proc)
    pltpu.sync_copy(acc0, s1p_hbm.at[q, pair])
    pltpu.sync_copy(acc1, s1p_hbm.at[q, pair + H // 2])


NF2 = 5
NQ2 = 4

@functools.partial(
    pl.kernel, mesh=_MESH,
    out_type=jax.ShapeDtypeStruct((NQ2, C, N), jnp.float32),
    scratch_types=(
        [pltpu.VMEM((N,), jnp.float32) for _ in range(2 * NF2)]
        + _edge_bufs()
    ),
    compiler_params=_SC_PARAMS,
)
def _l2_kernel(eif_hbm, ew_hbm, g2t_hbm, s2p_hbm, *scratch):
    hcols = scratch[:NF2]
    accs = scratch[NF2:2 * NF2]
    bufs = scratch[2 * NF2:]
    rowv = (bufs[0], bufs[4])
    colv = (bufs[1], bufs[5])
    eww = (bufs[2], bufs[6])
    sem = (bufs[3], bufs[7])
    wid = _wid()
    grp_id = wid // NQ2
    q = wid % NQ2
    epq = E // NQ2
    nch = epq // CH
    for k in range(NF2):
        _zero(accs[k])
        pltpu.sync_copy(g2t_hbm.at[grp_id * NF2 + k], hcols[k])

    def start(ch, b):
        base = q * epq + ch * CH
        pltpu.async_copy(eif_hbm.at[pl.ds(base, CH)], rowv[b], sem[b])
        pltpu.async_copy(eif_hbm.at[pl.ds(E + base, CH)], colv[b], sem[b])
        pltpu.async_copy(ew_hbm.at[pl.ds(base, CH)], eww[b], sem[b])

    def wait(b):
        pltpu.make_async_copy(eif_hbm.at[pl.ds(0, CH)], rowv[b], sem[b]).wait()
        pltpu.make_async_copy(eif_hbm.at[pl.ds(0, CH)], colv[b], sem[b]).wait()
        pltpu.make_async_copy(ew_hbm.at[pl.ds(0, CH)], eww[b], sem[b]).wait()

    def proc(ch, b):
        @plsc.parallel_loop(0, GRP, unroll=4)
        def _g(g):
            rvec = rowv[b][pl.ds(g * 16, 16)]
            cvec = colv[b][pl.ds(g * 16, 16)]
            wvec = eww[b][pl.ds(g * 16, 16)]
            for k in range(NF2):
                vals = plsc.load_gather(hcols[k], [rvec]) * wvec
                plsc.addupdate_scatter(accs[k], [cvec], vals)

    _pipelined_chunks(nch, start, wait, proc)
    for k in range(NF2):
        pltpu.sync_copy(accs[k], s2p_hbm.at[q, grp_id * NF2 + k])


def _tc1_body(x_ref, w1_ref, degp_ref, pk1t_ref, z1t_ref, dis_ref, inv_ref):
    deg = 1.0 + jnp.sum(degp_ref[...], axis=0, keepdims=True)
    dis = jnp.where(deg > 0.0, lax.rsqrt(deg), 0.0)
    inv = jnp.where(deg > 0.0, 1.0 / deg, 0.0)
    h1t = lax.dot_general(w1_ref[...], x_ref[...],
                          (((0,), (1,)), ((), ())),
                          preferred_element_type=jnp.float32)
    g1t = h1t * dis
    lo = lax.bitcast_convert_type(g1t[:H // 2].astype(jnp.bfloat16),
                                  jnp.uint16).astype(jnp.uint32)
    hi = lax.bitcast_convert_type(g1t[H // 2:].astype(jnp.bfloat16),
                                  jnp.uint16).astype(jnp.uint32)
    pk1t_ref[...] = lax.bitcast_convert_type(lo | (hi << 16), jnp.int32)
    z1t_ref[...] = h1t * inv
    dis_ref[...] = dis
    inv_ref[...] = inv


def _tc2_body(s1p_ref, z1t_ref, dis_ref, inv_ref, w2_ref, b1_ref,
              x1_ref, g2t_ref, z2t_ref):
    dis = dis_ref[...]
    x1t = jnp.maximum(dis * (s1p_ref[0] + s1p_ref[1] + s1p_ref[2]
                             + s1p_ref[3]) + z1t_ref[...]
                      + b1_ref[...], 0.0)
    h2t = lax.dot_general(w2_ref[...], x1t,
                          (((0,), (0,)), ((), ())),
                          preferred_element_type=jnp.float32)
    x1_ref[...] = x1t.T
    g2t_ref[...] = h2t * dis
    z2t_ref[...] = h2t * inv_ref[...]


def _tc3_body(s2p_ref, z2t_ref, dis_ref, b2_ref, out_ref):
    s = s2p_ref[0] + s2p_ref[1] + s2p_ref[2] + s2p_ref[3]
    out_ref[...] = (dis_ref[...] * s + z2t_ref[...] + b2_ref[...]).T


_tc1 = pl.pallas_call(
    _tc1_body,
    out_shape=[
        jax.ShapeDtypeStruct((H // 2, N), jnp.int32),
        jax.ShapeDtypeStruct((H, N), jnp.float32),
        jax.ShapeDtypeStruct((1, N), jnp.float32),
        jax.ShapeDtypeStruct((1, N), jnp.float32),
    ],
)

_tc2 = pl.pallas_call(
    _tc2_body,
    out_shape=[
        jax.ShapeDtypeStruct((N, H), jnp.float32),
        jax.ShapeDtypeStruct((C, N), jnp.float32),
        jax.ShapeDtypeStruct((C, N), jnp.float32),
    ],
)

_tc3 = pl.pallas_call(
    _tc3_body,
    out_shape=jax.ShapeDtypeStruct((N, C), jnp.float32),
)


@jax.jit
def kernel(x, edge_index, edge_weight, W1, b1, W2, b2):
    eif = edge_index.reshape(-1)
    degp = _deg_kernel(eif, edge_weight)
    pk1t, z1t, dis, inv = _tc1(x, W1, degp)
    s1p = _l1_kernel(eif, edge_weight, pk1t)
    x1, g2t, z2t = _tc2(s1p, z1t, dis, inv, W2, b1[:, None])
    s2p = _l2_kernel(eif, edge_weight, g2t)
    out = _tc3(s2p, z2t, dis, b2[:, None])
    return (out, x1)


# speedup vs baseline: 1.0663x; 1.0359x over previous
import functools

import jax
import jax.numpy as jnp
from jax import lax
from jax.experimental import pallas as pl
from jax.experimental.pallas import tpu as pltpu
from jax.experimental.pallas import tpu_sc as plsc

N = 10000
E = 320000
F_IN = 128
H = 16
C = 40

NTILES = 32
CH = 3200
GRP = CH // 16

_SC_PARAMS = pltpu.CompilerParams(needs_layout_passes=False)
_MESH = plsc.VectorSubcoreMesh(core_axis_name="c", subcore_axis_name="s")


def _wid():
    return lax.axis_index("s") * 2 + lax.axis_index("c")


def _zero(ref):
    @plsc.parallel_loop(0, N // 16, unroll=8)
    def _(i):
        ref[pl.ds(i * 16, 16)] = jnp.zeros((16,), jnp.float32)


def _edge_bufs(with_row=True):
    if with_row:
        per_par = [
            pltpu.VMEM((2, CH), jnp.int32),
            pltpu.VMEM((CH,), jnp.float32),
            pltpu.SemaphoreType.DMA,
        ]
    else:
        per_par = [
            pltpu.VMEM((CH,), jnp.int32),
            pltpu.VMEM((CH,), jnp.float32),
            pltpu.SemaphoreType.DMA,
        ]
    return per_par + per_par


def _pipelined_chunks(nch, start_fn, wait_fn, proc_fn):
    start_fn(0, 0)

    def pair(p, _):
        e = 2 * p
        start_fn(e + 1, 1)
        wait_fn(0)
        proc_fn(e, 0)

        @pl.when(e + 2 < nch)
        def _b():
            start_fn(e + 2, 0)

        wait_fn(1)
        proc_fn(e + 1, 1)
        return _

    lax.fori_loop(0, nch // 2, pair, 0)
    if nch % 2:
        wait_fn(0)
        proc_fn(nch - 1, 0)


@functools.partial(
    pl.kernel, mesh=_MESH,
    out_type=jax.ShapeDtypeStruct((NTILES, N), jnp.float32),
    scratch_types=[
        pltpu.VMEM((N,), jnp.float32),
        pltpu.VMEM((2, CH), jnp.int32),
        pltpu.VMEM((CH,), jnp.float32),
        pltpu.SemaphoreType.DMA,
    ],
    compiler_params=_SC_PARAMS,
)
def _deg_kernel(ei_hbm, ew_hbm, degp_hbm, acc, rcv, eww, sem):
    wid = _wid()
    _zero(acc)
    ncht = E // CH

    def chunk(j, _):
        c = wid + NTILES * j

        @pl.when(c < ncht)
        def _p():
            base = c * CH
            pltpu.async_copy(ei_hbm.at[:, pl.ds(base, CH)], rcv, sem)
            pltpu.async_copy(ew_hbm.at[pl.ds(base, CH)], eww, sem)
            pltpu.make_async_copy(ei_hbm.at[:, pl.ds(0, CH)], rcv, sem).wait()
            pltpu.make_async_copy(ew_hbm.at[pl.ds(0, CH)], eww, sem).wait()

            @plsc.parallel_loop(0, GRP, unroll=8)
            def _g(g):
                cvec = rcv[1, pl.ds(g * 16, 16)]
                wvec = eww[pl.ds(g * 16, 16)]
                plsc.addupdate_scatter(acc, [cvec], wvec)
        return _

    lax.fori_loop(0, (E // CH + NTILES - 1) // NTILES, chunk, 0)
    pltpu.sync_copy(acc, degp_hbm.at[wid])


@functools.partial(
    pl.kernel, mesh=_MESH,
    out_type=jax.ShapeDtypeStruct((4, H, N), jnp.float32),
    scratch_types=[
        pltpu.VMEM((N,), jnp.int32),
        pltpu.VMEM((N,), jnp.float32),
        pltpu.VMEM((N,), jnp.float32),
    ] + _edge_bufs(),
    compiler_params=_SC_PARAMS,
)
def _l1_kernel(ei_hbm, ew_hbm, pk1t_hbm, s1p_hbm,
               pkcol, acc0, acc1, *bufs):
    rcv = (bufs[0], bufs[3])
    eww = (bufs[1], bufs[4])
    sem = (bufs[2], bufs[5])
    wid = _wid()
    pair = wid // 4
    q = wid % 4
    epq = E // 4
    nch = epq // CH
    _zero(acc0)
    _zero(acc1)
    pltpu.sync_copy(pk1t_hbm.at[pair], pkcol)

    def start(ch, b):
        base = q * epq + ch * CH
        pltpu.async_copy(ei_hbm.at[:, pl.ds(base, CH)], rcv[b], sem[b])
        pltpu.async_copy(ew_hbm.at[pl.ds(base, CH)], eww[b], sem[b])

    def wait(b):
        pltpu.make_async_copy(ei_hbm.at[:, pl.ds(0, CH)], rcv[b], sem[b]).wait()
        pltpu.make_async_copy(ew_hbm.at[pl.ds(0, CH)], eww[b], sem[b]).wait()

    def proc(ch, b):
        @plsc.parallel_loop(0, GRP, unroll=8)
        def _g(g):
            rvec = rcv[b][0, pl.ds(g * 16, 16)]
            cvec = rcv[b][1, pl.ds(g * 16, 16)]
            wvec = eww[b][pl.ds(g * 16, 16)]
            word = plsc.load_gather(pkcol, [rvec])
            v0 = plsc.bitcast(lax.shift_left(word, 16), jnp.float32)
            v1 = plsc.bitcast(word & jnp.int32(-65536), jnp.float32)
            plsc.addupdate_scatter(acc0, [cvec], v0 * wvec)
            plsc.addupdate_scatter(acc1, [cvec], v1 * wvec)

    _pipelined_chunks(nch, start, wait, proc)
    pltpu.sync_copy(acc0, s1p_hbm.at[q, pair])
    pltpu.sync_copy(acc1, s1p_hbm.at[q, pair + H // 2])


NF2 = 5
NQ2 = 4

@functools.partial(
    pl.kernel, mesh=_MESH,
    out_type=jax.ShapeDtypeStruct((NQ2, C, N), jnp.float32),
    scratch_types=(
        [pltpu.VMEM((N,), jnp.float32) for _ in range(2 * NF2)]
        + _edge_bufs()
    ),
    compiler_params=_SC_PARAMS,
)
def _l2_kernel(ei_hbm, ew_hbm, g2t_hbm, s2p_hbm, *scratch):
    hcols = scratch[:NF2]
    accs = scratch[NF2:2 * NF2]
    bufs = scratch[2 * NF2:]
    rcv = (bufs[0], bufs[3])
    eww = (bufs[1], bufs[4])
    sem = (bufs[2], bufs[5])
    wid = _wid()
    grp_id = wid // NQ2
    q = wid % NQ2
    epq = E // NQ2
    nch = epq // CH
    for k in range(NF2):
        _zero(accs[k])
        pltpu.sync_copy(g2t_hbm.at[grp_id * NF2 + k], hcols[k])

    def start(ch, b):
        base = q * epq + ch * CH
        pltpu.async_copy(ei_hbm.at[:, pl.ds(base, CH)], rcv[b], sem[b])
        pltpu.async_copy(ew_hbm.at[pl.ds(base, CH)], eww[b], sem[b])

    def wait(b):
        pltpu.make_async_copy(ei_hbm.at[:, pl.ds(0, CH)], rcv[b], sem[b]).wait()
        pltpu.make_async_copy(ew_hbm.at[pl.ds(0, CH)], eww[b], sem[b]).wait()

    def proc(ch, b):
        @plsc.parallel_loop(0, GRP, unroll=4)
        def _g(g):
            rvec = rcv[b][0, pl.ds(g * 16, 16)]
            cvec = rcv[b][1, pl.ds(g * 16, 16)]
            wvec = eww[b][pl.ds(g * 16, 16)]
            for k in range(NF2):
                vals = plsc.load_gather(hcols[k], [rvec]) * wvec
                plsc.addupdate_scatter(accs[k], [cvec], vals)

    _pipelined_chunks(nch, start, wait, proc)
    for k in range(NF2):
        pltpu.sync_copy(accs[k], s2p_hbm.at[q, grp_id * NF2 + k])


def _tc1_body(x_ref, w1_ref, degp_ref, pk1t_ref, z1t_ref, dis_ref, inv_ref):
    deg = 1.0 + jnp.sum(degp_ref[...], axis=0, keepdims=True)
    dis = jnp.where(deg > 0.0, lax.rsqrt(deg), 0.0)
    inv = jnp.where(deg > 0.0, 1.0 / deg, 0.0)
    h1t = lax.dot_general(w1_ref[...], x_ref[...],
                          (((0,), (1,)), ((), ())),
                          preferred_element_type=jnp.float32)
    g1t = h1t * dis
    lo = lax.bitcast_convert_type(g1t[:H // 2].astype(jnp.bfloat16),
                                  jnp.uint16).astype(jnp.uint32)
    hi = lax.bitcast_convert_type(g1t[H // 2:].astype(jnp.bfloat16),
                                  jnp.uint16).astype(jnp.uint32)
    pk1t_ref[...] = lax.bitcast_convert_type(lo | (hi << 16), jnp.int32)
    z1t_ref[...] = h1t * inv
    dis_ref[...] = dis
    inv_ref[...] = inv


def _tc2_body(s1p_ref, z1t_ref, dis_ref, inv_ref, w2_ref, b1_ref,
              x1_ref, g2t_ref, z2t_ref):
    dis = dis_ref[...]
    x1t = jnp.maximum(dis * (s1p_ref[0] + s1p_ref[1] + s1p_ref[2]
                             + s1p_ref[3]) + z1t_ref[...]
                      + b1_ref[...], 0.0)
    h2t = lax.dot_general(w2_ref[...], x1t,
                          (((0,), (0,)), ((), ())),
                          preferred_element_type=jnp.float32)
    x1_ref[...] = x1t.T
    g2t_ref[...] = h2t * dis
    z2t_ref[...] = h2t * inv_ref[...]


def _tc3_body(s2p_ref, z2t_ref, dis_ref, b2_ref, out_ref):
    s = s2p_ref[0] + s2p_ref[1] + s2p_ref[2] + s2p_ref[3]
    out_ref[...] = (dis_ref[...] * s + z2t_ref[...] + b2_ref[...]).T


_tc1 = pl.pallas_call(
    _tc1_body,
    out_shape=[
        jax.ShapeDtypeStruct((H // 2, N), jnp.int32),
        jax.ShapeDtypeStruct((H, N), jnp.float32),
        jax.ShapeDtypeStruct((1, N), jnp.float32),
        jax.ShapeDtypeStruct((1, N), jnp.float32),
    ],
)

_tc2 = pl.pallas_call(
    _tc2_body,
    out_shape=[
        jax.ShapeDtypeStruct((N, H), jnp.float32),
        jax.ShapeDtypeStruct((C, N), jnp.float32),
        jax.ShapeDtypeStruct((C, N), jnp.float32),
    ],
)

_tc3 = pl.pallas_call(
    _tc3_body,
    out_shape=jax.ShapeDtypeStruct((N, C), jnp.float32),
)


@jax.jit
def kernel(x, edge_index, edge_weight, W1, b1, W2, b2):
    degp = _deg_kernel(edge_index, edge_weight)
    pk1t, z1t, dis, inv = _tc1(x, W1, degp)
    s1p = _l1_kernel(edge_index, edge_weight, pk1t)
    x1, g2t, z2t = _tc2(s1p, z1t, dis, inv, W2, b1[:, None])
    s2p = _l2_kernel(edge_index, edge_weight, g2t)
    out = _tc3(s2p, z2t, dis, b2[:, None])
    return (out, x1)


# speedup vs baseline: 1.0677x; 1.0013x over previous
import functools

import jax
import jax.numpy as jnp
from jax import lax
from jax.experimental import pallas as pl
from jax.experimental.pallas import tpu as pltpu
from jax.experimental.pallas import tpu_sc as plsc

N = 10000
E = 320000
F_IN = 128
H = 16
C = 40

NTILES = 32
CH = 3200
GRP = CH // 16

_SC_PARAMS = pltpu.CompilerParams(needs_layout_passes=False)
_MESH = plsc.VectorSubcoreMesh(core_axis_name="c", subcore_axis_name="s")


def _wid():
    return lax.axis_index("s") * 2 + lax.axis_index("c")


def _zero(ref):
    @plsc.parallel_loop(0, N // 16, unroll=8)
    def _(i):
        ref[pl.ds(i * 16, 16)] = jnp.zeros((16,), jnp.float32)


def _edge_bufs(with_row=True):
    if with_row:
        per_par = [
            pltpu.VMEM((2, CH), jnp.int32),
            pltpu.VMEM((CH,), jnp.float32),
            pltpu.SemaphoreType.DMA,
        ]
    else:
        per_par = [
            pltpu.VMEM((CH,), jnp.int32),
            pltpu.VMEM((CH,), jnp.float32),
            pltpu.SemaphoreType.DMA,
        ]
    return per_par + per_par


def _pipelined_chunks(nch, start_fn, wait_fn, proc_fn):
    start_fn(0, 0)

    def pair(p, _):
        e = 2 * p
        start_fn(e + 1, 1)
        wait_fn(0)
        proc_fn(e, 0)

        @pl.when(e + 2 < nch)
        def _b():
            start_fn(e + 2, 0)

        wait_fn(1)
        proc_fn(e + 1, 1)
        return _

    lax.fori_loop(0, nch // 2, pair, 0)
    if nch % 2:
        wait_fn(0)
        proc_fn(nch - 1, 0)


@functools.partial(
    pl.kernel, mesh=_MESH,
    out_type=jax.ShapeDtypeStruct((NTILES, N), jnp.float32),
    scratch_types=[
        pltpu.VMEM((N,), jnp.float32),
        pltpu.VMEM((2, CH), jnp.int32),
        pltpu.VMEM((CH,), jnp.float32),
        pltpu.SemaphoreType.DMA,
    ],
    compiler_params=_SC_PARAMS,
)
def _deg_kernel(ei_hbm, ew_hbm, degp_hbm, acc, rcv, eww, sem):
    wid = _wid()
    _zero(acc)
    ncht = E // CH

    def chunk(j, _):
        c = wid + NTILES * j

        @pl.when(c < ncht)
        def _p():
            base = c * CH
            pltpu.async_copy(ei_hbm.at[:, pl.ds(base, CH)], rcv, sem)
            pltpu.async_copy(ew_hbm.at[pl.ds(base, CH)], eww, sem)
            pltpu.make_async_copy(ei_hbm.at[:, pl.ds(0, CH)], rcv, sem).wait()
            pltpu.make_async_copy(ew_hbm.at[pl.ds(0, CH)], eww, sem).wait()

            @plsc.parallel_loop(0, GRP, unroll=8)
            def _g(g):
                cvec = rcv[1, pl.ds(g * 16, 16)]
                wvec = eww[pl.ds(g * 16, 16)]
                plsc.addupdate_scatter(acc, [cvec], wvec)
        return _

    lax.fori_loop(0, (E // CH + NTILES - 1) // NTILES, chunk, 0)
    pltpu.sync_copy(acc, degp_hbm.at[wid])


@functools.partial(
    pl.kernel, mesh=_MESH,
    out_type=jax.ShapeDtypeStruct((4, H, N), jnp.float32),
    scratch_types=[
        pltpu.VMEM((N,), jnp.int32),
        pltpu.VMEM((N,), jnp.float32),
        pltpu.VMEM((N,), jnp.float32),
    ] + _edge_bufs(),
    compiler_params=_SC_PARAMS,
)
def _l1_kernel(ei_hbm, ew_hbm, pk1t_hbm, s1p_hbm,
               pkcol, acc0, acc1, *bufs):
    rcv = (bufs[0], bufs[3])
    eww = (bufs[1], bufs[4])
    sem = (bufs[2], bufs[5])
    wid = _wid()
    pair = wid // 4
    q = wid % 4
    epq = E // 4
    nch = epq // CH
    _zero(acc0)
    _zero(acc1)
    pltpu.sync_copy(pk1t_hbm.at[pair], pkcol)

    def start(ch, b):
        base = q * epq + ch * CH
        pltpu.async_copy(ei_hbm.at[:, pl.ds(base, CH)], rcv[b], sem[b])
        pltpu.async_copy(ew_hbm.at[pl.ds(base, CH)], eww[b], sem[b])

    def wait(b):
        pltpu.make_async_copy(ei_hbm.at[:, pl.ds(0, CH)], rcv[b], sem[b]).wait()
        pltpu.make_async_copy(ew_hbm.at[pl.ds(0, CH)], eww[b], sem[b]).wait()

    def proc(ch, b):
        @plsc.parallel_loop(0, GRP, unroll=4)
        def _g(g):
            rvec = rcv[b][0, pl.ds(g * 16, 16)]
            cvec = rcv[b][1, pl.ds(g * 16, 16)]
            wvec = eww[b][pl.ds(g * 16, 16)]
            word = plsc.load_gather(pkcol, [rvec])
            v0 = plsc.bitcast(lax.shift_left(word, 16), jnp.float32)
            v1 = plsc.bitcast(word & jnp.int32(-65536), jnp.float32)
            plsc.addupdate_scatter(acc0, [cvec], v0 * wvec)
            plsc.addupdate_scatter(acc1, [cvec], v1 * wvec)

    _pipelined_chunks(nch, start, wait, proc)
    pltpu.sync_copy(acc0, s1p_hbm.at[q, pair])
    pltpu.sync_copy(acc1, s1p_hbm.at[q, pair + H // 2])


NF2 = 5
NQ2 = 4

@functools.partial(
    pl.kernel, mesh=_MESH,
    out_type=jax.ShapeDtypeStruct((NQ2, C, N), jnp.float32),
    scratch_types=(
        [pltpu.VMEM((N,), jnp.float32) for _ in range(2 * NF2)]
        + _edge_bufs()
    ),
    compiler_params=_SC_PARAMS,
)
def _l2_kernel(ei_hbm, ew_hbm, g2t_hbm, s2p_hbm, *scratch):
    hcols = scratch[:NF2]
    accs = scratch[NF2:2 * NF2]
    bufs = scratch[2 * NF2:]
    rcv = (bufs[0], bufs[3])
    eww = (bufs[1], bufs[4])
    sem = (bufs[2], bufs[5])
    wid = _wid()
    grp_id = wid // NQ2
    q = wid % NQ2
    epq = E // NQ2
    nch = epq // CH
    for k in range(NF2):
        _zero(accs[k])
        pltpu.sync_copy(g2t_hbm.at[grp_id * NF2 + k], hcols[k])

    def start(ch, b):
        base = q * epq + ch * CH
        pltpu.async_copy(ei_hbm.at[:, pl.ds(base, CH)], rcv[b], sem[b])
        pltpu.async_copy(ew_hbm.at[pl.ds(base, CH)], eww[b], sem[b])

    def wait(b):
        pltpu.make_async_copy(ei_hbm.at[:, pl.ds(0, CH)], rcv[b], sem[b]).wait()
        pltpu.make_async_copy(ew_hbm.at[pl.ds(0, CH)], eww[b], sem[b]).wait()

    def proc(ch, b):
        @plsc.parallel_loop(0, GRP, unroll=4)
        def _g(g):
            rvec = rcv[b][0, pl.ds(g * 16, 16)]
            cvec = rcv[b][1, pl.ds(g * 16, 16)]
            wvec = eww[b][pl.ds(g * 16, 16)]
            for k in range(NF2):
                vals = plsc.load_gather(hcols[k], [rvec]) * wvec
                plsc.addupdate_scatter(accs[k], [cvec], vals)

    _pipelined_chunks(nch, start, wait, proc)
    for k in range(NF2):
        pltpu.sync_copy(accs[k], s2p_hbm.at[q, grp_id * NF2 + k])


def _tc1_body(x_ref, w1_ref, degp_ref, pk1t_ref, z1t_ref, dis_ref, inv_ref):
    deg = 1.0 + jnp.sum(degp_ref[...], axis=0, keepdims=True)
    dis = jnp.where(deg > 0.0, lax.rsqrt(deg), 0.0)
    inv = jnp.where(deg > 0.0, 1.0 / deg, 0.0)
    h1t = lax.dot_general(w1_ref[...], x_ref[...],
                          (((0,), (1,)), ((), ())),
                          preferred_element_type=jnp.float32)
    g1t = h1t * dis
    lo = lax.bitcast_convert_type(g1t[:H // 2].astype(jnp.bfloat16),
                                  jnp.uint16).astype(jnp.uint32)
    hi = lax.bitcast_convert_type(g1t[H // 2:].astype(jnp.bfloat16),
                                  jnp.uint16).astype(jnp.uint32)
    pk1t_ref[...] = lax.bitcast_convert_type(lo | (hi << 16), jnp.int32)
    z1t_ref[...] = h1t * inv
    dis_ref[...] = dis
    inv_ref[...] = inv


def _tc2_body(s1p_ref, z1t_ref, dis_ref, inv_ref, w2_ref, b1_ref,
              x1_ref, g2t_ref, z2t_ref):
    dis = dis_ref[...]
    x1t = jnp.maximum(dis * (s1p_ref[0] + s1p_ref[1] + s1p_ref[2]
                             + s1p_ref[3]) + z1t_ref[...]
                      + b1_ref[...], 0.0)
    h2t = lax.dot_general(w2_ref[...], x1t,
                          (((0,), (0,)), ((), ())),
                          preferred_element_type=jnp.float32)
    x1_ref[...] = x1t.T
    g2t_ref[...] = h2t * dis
    z2t_ref[...] = h2t * inv_ref[...]


def _tc3_body(s2p_ref, z2t_ref, dis_ref, b2_ref, out_ref):
    s = s2p_ref[0] + s2p_ref[1] + s2p_ref[2] + s2p_ref[3]
    out_ref[...] = (dis_ref[...] * s + z2t_ref[...] + b2_ref[...]).T


_tc1 = pl.pallas_call(
    _tc1_body,
    out_shape=[
        jax.ShapeDtypeStruct((H // 2, N), jnp.int32),
        jax.ShapeDtypeStruct((H, N), jnp.float32),
        jax.ShapeDtypeStruct((1, N), jnp.float32),
        jax.ShapeDtypeStruct((1, N), jnp.float32),
    ],
)

_tc2 = pl.pallas_call(
    _tc2_body,
    out_shape=[
        jax.ShapeDtypeStruct((N, H), jnp.float32),
        jax.ShapeDtypeStruct((C, N), jnp.float32),
        jax.ShapeDtypeStruct((C, N), jnp.float32),
    ],
)

_tc3 = pl.pallas_call(
    _tc3_body,
    out_shape=jax.ShapeDtypeStruct((N, C), jnp.float32),
)


@jax.jit
def kernel(x, edge_index, edge_weight, W1, b1, W2, b2):
    degp = _deg_kernel(edge_index, edge_weight)
    pk1t, z1t, dis, inv = _tc1(x, W1, degp)
    s1p = _l1_kernel(edge_index, edge_weight, pk1t)
    x1, g2t, z2t = _tc2(s1p, z1t, dis, inv, W2, b1[:, None])
    s2p = _l2_kernel(edge_index, edge_weight, g2t)
    out = _tc3(s2p, z2t, dis, b2[:, None])
    return (out, x1)
